# Initial kernel scaffold; baseline (speedup 1.0000x reference)
import jax
import jax.numpy as jnp
from jax.experimental import pallas as pl


def kernel(X_w, X_d, tables, W1, b1, W2, b2, Wout, bout):
    raise NotImplementedError("write your pallas kernel here")


# speedup vs baseline: 2.4093x; 2.4093x over previous
import functools

import jax
import jax.numpy as jnp
from jax import lax
from jax.experimental import pallas as pl
from jax.experimental.pallas import tpu as pltpu
from jax.experimental.pallas import tpu_sc as plsc

_B = 16384
_NF = 26
_NC = 13
_V = 100000
_D = 16
_WD = 128
_H1 = 256
_H2 = 128
_NCLS = 2
_DE = _NF * _D

_NCORES = 2
_NSUB = 16
_NW = _NCORES * _NSUB
_IDX_PW = _B * _NF // _NW
_SPW = 128
_NSTREAM = _IDX_PW // _SPW
_GROUP = 13
_NGROUP = _NSTREAM // _GROUP
_CHUNK = _GROUP * _SPW


def _sc_gather_body(idx_hbm, table_hbm, out_hbm, idx_v, rows_v, sem):
    wid = lax.axis_index("s") * _NCORES + lax.axis_index("c")
    pltpu.sync_copy(idx_hbm.at[pl.ds(wid * _NSTREAM, _NSTREAM), :], idx_v)
    out_base = wid * _IDX_PW

    def group(g, carry):
        handles = []
        for j in range(_GROUP):
            handles.append(pltpu.async_copy(
                table_hbm.at[idx_v.at[g * _GROUP + j]],
                rows_v.at[pl.ds(j * _SPW, _SPW), :],
                sem))
        for h in handles:
            h.wait()
        pltpu.sync_copy(rows_v,
                        out_hbm.at[pl.ds(out_base + g * _CHUNK, _CHUNK), :])
        return carry

    lax.fori_loop(0, _NGROUP, group, 0)


@functools.cache
def _sc_gather():
    return pl.kernel(
        _sc_gather_body,
        out_type=jax.ShapeDtypeStruct((_B * _NF, _D), jnp.float32),
        mesh=plsc.VectorSubcoreMesh(core_axis_name="c", subcore_axis_name="s",
                                    num_cores=_NCORES, num_subcores=_NSUB),
        scratch_types=[
            pltpu.VMEM((_NSTREAM, _SPW), jnp.int32),
            pltpu.VMEM((_CHUNK, _D), jnp.float32),
            pltpu.SemaphoreType.DMA,
        ],
        compiler_params=pltpu.CompilerParams(use_tc_tiling_on_sc=False),
    )


def _mlp_body(xd_ref, emb_ref, xw_ref, w1a_ref, w1e_ref, r_ref, b1_ref,
              w2_ref, b2_ref, wd_ref, ww_ref, bo_ref, out_ref):
    xd = xd_ref[...]
    cont = xd[:, _NF:].astype(jnp.float32)
    mask = (xd[:, :_NF] == 0).astype(jnp.float32)
    cneg = -jnp.dot(r_ref[...], w1a_ref[...],
                    preferred_element_type=jnp.float32)
    x1 = jnp.dot(emb_ref[...], w1a_ref[...],
                 preferred_element_type=jnp.float32)
    x1 = x1 + jnp.dot(cont, w1e_ref[...], preferred_element_type=jnp.float32)
    x1 = x1 + jnp.dot(mask, cneg, preferred_element_type=jnp.float32)
    x1 = jnp.maximum(x1 + b1_ref[...], 0.0)
    x2 = jnp.dot(x1, w2_ref[...], preferred_element_type=jnp.float32)
    x2 = jnp.maximum(x2 + b2_ref[...], 0.0)
    o = jnp.dot(x2, wd_ref[...], preferred_element_type=jnp.float32)
    o = o + jnp.dot(xw_ref[...], ww_ref[...], preferred_element_type=jnp.float32)
    out_ref[...] = o + bo_ref[...]


_BM = 2048


def _mlp_call(interpret=False):
    full = lambda a, b: pl.BlockSpec((a, b), lambda i: (0, 0))
    return pl.pallas_call(
        _mlp_body,
        grid=(_B // _BM,),
        in_specs=[
            pl.BlockSpec((_BM, _NF + _NC), lambda i: (i, 0)),
            pl.BlockSpec((_BM, _DE), lambda i: (i, 0)),
            pl.BlockSpec((_BM, _WD), lambda i: (i, 0)),
            full(_DE, _H1),
            full(_NC, _H1),
            full(_NF, _DE),
            full(1, _H1),
            full(_H1, _H2),
            full(1, _H2),
            full(_H2, _NCLS),
            full(_WD, _NCLS),
            full(1, _NCLS),
        ],
        out_specs=pl.BlockSpec((_BM, _NCLS), lambda i: (i, 0)),
        out_shape=jax.ShapeDtypeStruct((_B, _NCLS), jnp.float32),
        interpret=interpret,
    )


def kernel(X_w, X_d, tables, W1, b1, W2, b2, Wout, bout):
    table_flat = tables.reshape(_NF * _V, _D)
    offs = (jnp.arange(_NF, dtype=jnp.int32) * _V)[None, :]
    idx_flat = (X_d[:, :_NF].astype(jnp.int32) + offs).reshape(
        _B * _NF // _SPW, _SPW)
    emb = _sc_gather()(idx_flat, table_flat).reshape(_B, _DE)

    rows0 = tables[:, 0, :]
    R = (jnp.eye(_NF, dtype=jnp.float32)[:, :, None]
         * rows0[:, None, :]).reshape(_NF, _DE)
    out = _mlp_call()(
        X_d, emb, X_w.astype(jnp.float32),
        W1[:_DE], W1[_DE:], R,
        b1.reshape(1, _H1), W2, b2.reshape(1, _H2),
        Wout[:_H2], Wout[_H2:], bout.reshape(1, _NCLS))
    return out


# speedup vs baseline: 11.6782x; 4.8471x over previous
import functools

import jax
import jax.numpy as jnp
from jax import lax
from jax.experimental import pallas as pl
from jax.experimental.pallas import tpu as pltpu
from jax.experimental.pallas import tpu_sc as plsc

_B = 16384
_NF = 26
_NC = 13
_V = 100000
_D = 16
_WD = 128
_H1 = 256
_H2 = 128
_NCLS = 2
_DE = _NF * _D

_NCORES = 2
_NSUB = 16
_NW = _NCORES * _NSUB
_PPW = _DE // _NW
_BCH = 8192
_NBCH = _B // _BCH


def _sc_plane_gather_body(t2_hbm, idxT_hbm, out_hbm, plane_v, iv_v, ob_v):
    wid = lax.axis_index("s") * _NCORES + lax.axis_index("c")

    def plane_step(p, carry):
        plane = wid * _PPW + p
        f = plane // _D
        d = plane % _D
        pltpu.sync_copy(t2_hbm.at[f, d, :], plane_v)

        def bchunk(c, carry2):
            pltpu.sync_copy(idxT_hbm.at[f, pl.ds(c * _BCH, _BCH)], iv_v)

            def step(i, carry3):
                ivv = iv_v[pl.ds(i * 16, 16)]
                ob_v[pl.ds(i * 16, 16)] = plsc.load_gather(plane_v, [ivv])
                return carry3

            lax.fori_loop(0, _BCH // 16, step, 0)
            pltpu.sync_copy(ob_v, out_hbm.at[plane, pl.ds(c * _BCH, _BCH)])
            return carry2

        lax.fori_loop(0, _NBCH, bchunk, 0)
        return carry

    lax.fori_loop(0, _PPW, plane_step, 0)


@functools.cache
def _sc_gather():
    return pl.kernel(
        _sc_plane_gather_body,
        out_type=jax.ShapeDtypeStruct((_DE, _B), jnp.float32),
        mesh=plsc.VectorSubcoreMesh(core_axis_name="c", subcore_axis_name="s",
                                    num_cores=_NCORES, num_subcores=_NSUB),
        scratch_types=[
            pltpu.VMEM((_V,), jnp.float32),
            pltpu.VMEM((_BCH,), jnp.int32),
            pltpu.VMEM((_BCH,), jnp.float32),
        ],
        compiler_params=pltpu.CompilerParams(use_tc_tiling_on_sc=True,
                                             needs_layout_passes=False),
    )


def _mlp_body(xdT_ref, embT_ref, xwT_ref, w1aT_ref, w1eT_ref, rT_ref, b1_ref,
              w2T_ref, b2_ref, wdT_ref, wwT_ref, boT_ref, outT_ref):
    xdT = xdT_ref[...]
    contT = xdT[_NF:, :].astype(jnp.float32)
    maskT = (xdT[:_NF, :] == 0).astype(jnp.float32)
    cnegT = -jnp.dot(w1aT_ref[...], rT_ref[...],
                     preferred_element_type=jnp.float32)
    x1 = jnp.dot(w1aT_ref[...], embT_ref[...],
                 preferred_element_type=jnp.float32)
    x1 = x1 + jnp.dot(w1eT_ref[...], contT, preferred_element_type=jnp.float32)
    x1 = x1 + jnp.dot(cnegT, maskT, preferred_element_type=jnp.float32)
    x1 = jnp.maximum(x1 + b1_ref[...], 0.0)
    x2 = jnp.dot(w2T_ref[...], x1, preferred_element_type=jnp.float32)
    x2 = jnp.maximum(x2 + b2_ref[...], 0.0)
    o = jnp.dot(wdT_ref[...], x2, preferred_element_type=jnp.float32)
    o = o + jnp.dot(wwT_ref[...], xwT_ref[...],
                    preferred_element_type=jnp.float32)
    outT_ref[...] = o + boT_ref[...]


_BM = 2048


def _mlp_call(interpret=False):
    full = lambda a, b: pl.BlockSpec((a, b), lambda i: (0, 0))
    return pl.pallas_call(
        _mlp_body,
        grid=(_B // _BM,),
        in_specs=[
            pl.BlockSpec((_NF + _NC, _BM), lambda i: (0, i)),
            pl.BlockSpec((_DE, _BM), lambda i: (0, i)),
            pl.BlockSpec((_WD, _BM), lambda i: (0, i)),
            full(_H1, _DE),
            full(_H1, _NC),
            full(_DE, _NF),
            full(_H1, 1),
            full(_H2, _H1),
            full(_H2, 1),
            full(_NCLS, _H2),
            full(_NCLS, _WD),
            full(_NCLS, 1),
        ],
        out_specs=pl.BlockSpec((_NCLS, _BM), lambda i: (0, i)),
        out_shape=jax.ShapeDtypeStruct((_NCLS, _B), jnp.float32),
        interpret=interpret,
    )


def kernel(X_w, X_d, tables, W1, b1, W2, b2, Wout, bout):
    t2 = jnp.transpose(tables, (0, 2, 1))
    idxT = X_d[:, :_NF].astype(jnp.int32).T
    embT = _sc_gather()(t2, idxT)

    rows0 = tables[:, 0, :]
    R = (jnp.eye(_NF, dtype=jnp.float32)[:, :, None]
         * rows0[:, None, :]).reshape(_NF, _DE)
    outT = _mlp_call()(
        X_d.T, embT, X_w.astype(jnp.float32).T,
        W1[:_DE].T, W1[_DE:].T, R.T,
        b1.reshape(_H1, 1), W2.T, b2.reshape(_H2, 1),
        Wout[:_H2].T, Wout[_H2:].T, bout.reshape(_NCLS, 1))
    return outT.T


# speedup vs baseline: 14.9535x; 1.2805x over previous
import functools

import jax
import jax.numpy as jnp
from jax import lax
from jax.experimental import pallas as pl
from jax.experimental.pallas import tpu as pltpu
from jax.experimental.pallas import tpu_sc as plsc

_B = 16384
_NF = 26
_NC = 13
_V = 100000
_D = 16
_WD = 128
_H1 = 256
_H2 = 128
_NCLS = 2
_DE = _NF * _D

_NCORES = 2
_NSUB = 16
_NW = _NCORES * _NSUB
_PPW = _DE // _NW
_BCH = 8192
_NBCH = _B // _BCH


_UNR = 8


def _sc_plane_gather_body(t2_hbm, idxT_hbm, out_hbm, plane_v, iv_v, ob_v):
    wid = lax.axis_index("s") * _NCORES + lax.axis_index("c")

    def plane_step(p, prev_f):
        plane = wid * _PPW + p
        f = plane // _D
        d = plane % _D

        @pl.when(f != prev_f)
        def _():
            pltpu.sync_copy(idxT_hbm.at[f, :], iv_v)

        pltpu.sync_copy(t2_hbm.at[f, d, :], plane_v)

        def bchunk(c, carry2):
            def step(i, carry3):
                base = i * (16 * _UNR)
                for j in range(_UNR):
                    o = base + j * 16
                    ivv = iv_v[pl.ds(c * _BCH + o, 16)]
                    ob_v[pl.ds(o, 16)] = plsc.load_gather(plane_v, [ivv])
                return carry3

            lax.fori_loop(0, _BCH // (16 * _UNR), step, 0)
            pltpu.sync_copy(ob_v, out_hbm.at[plane, pl.ds(c * _BCH, _BCH)])
            return carry2

        lax.fori_loop(0, _NBCH, bchunk, 0)
        return f

    lax.fori_loop(0, _PPW, plane_step, -1)


@functools.cache
def _sc_gather():
    return pl.kernel(
        _sc_plane_gather_body,
        out_type=jax.ShapeDtypeStruct((_DE, _B), jnp.float32),
        mesh=plsc.VectorSubcoreMesh(core_axis_name="c", subcore_axis_name="s",
                                    num_cores=_NCORES, num_subcores=_NSUB),
        scratch_types=[
            pltpu.VMEM((_V,), jnp.float32),
            pltpu.VMEM((_B,), jnp.int32),
            pltpu.VMEM((_BCH,), jnp.float32),
        ],
        compiler_params=pltpu.CompilerParams(use_tc_tiling_on_sc=True,
                                             needs_layout_passes=False),
    )


def _mlp_body(xdT_ref, embT_ref, xwT_ref, w1aT_ref, w1eT_ref, rT_ref, b1_ref,
              w2T_ref, b2_ref, wdT_ref, wwT_ref, boT_ref, outT_ref):
    xdT = xdT_ref[...]
    contT = xdT[_NF:, :].astype(jnp.float32)
    maskT = (xdT[:_NF, :] == 0).astype(jnp.float32)
    cnegT = -jnp.dot(w1aT_ref[...], rT_ref[...],
                     preferred_element_type=jnp.float32)
    x1 = jnp.dot(w1aT_ref[...], embT_ref[...],
                 preferred_element_type=jnp.float32)
    x1 = x1 + jnp.dot(w1eT_ref[...], contT, preferred_element_type=jnp.float32)
    x1 = x1 + jnp.dot(cnegT, maskT, preferred_element_type=jnp.float32)
    x1 = jnp.maximum(x1 + b1_ref[...], 0.0)
    x2 = jnp.dot(w2T_ref[...], x1, preferred_element_type=jnp.float32)
    x2 = jnp.maximum(x2 + b2_ref[...], 0.0)
    o = jnp.dot(wdT_ref[...], x2, preferred_element_type=jnp.float32)
    o = o + jnp.dot(wwT_ref[...], xwT_ref[...],
                    preferred_element_type=jnp.float32)
    outT_ref[...] = o + boT_ref[...]


_BM = 2048


def _mlp_call(interpret=False):
    full = lambda a, b: pl.BlockSpec((a, b), lambda i: (0, 0))
    return pl.pallas_call(
        _mlp_body,
        grid=(_B // _BM,),
        in_specs=[
            pl.BlockSpec((_NF + _NC, _BM), lambda i: (0, i)),
            pl.BlockSpec((_DE, _BM), lambda i: (0, i)),
            pl.BlockSpec((_WD, _BM), lambda i: (0, i)),
            full(_H1, _DE),
            full(_H1, _NC),
            full(_DE, _NF),
            full(_H1, 1),
            full(_H2, _H1),
            full(_H2, 1),
            full(_NCLS, _H2),
            full(_NCLS, _WD),
            full(_NCLS, 1),
        ],
        out_specs=pl.BlockSpec((_NCLS, _BM), lambda i: (0, i)),
        out_shape=jax.ShapeDtypeStruct((_NCLS, _B), jnp.float32),
        interpret=interpret,
    )


def kernel(X_w, X_d, tables, W1, b1, W2, b2, Wout, bout):
    t2 = jnp.transpose(tables, (0, 2, 1))
    idxT = X_d[:, :_NF].astype(jnp.int32).T
    embT = _sc_gather()(t2, idxT)

    rows0 = tables[:, 0, :]
    R = (jnp.eye(_NF, dtype=jnp.float32)[:, :, None]
         * rows0[:, None, :]).reshape(_NF, _DE)
    outT = _mlp_call()(
        X_d.T, embT, X_w.astype(jnp.float32).T,
        W1[:_DE].T, W1[_DE:].T, R.T,
        b1.reshape(_H1, 1), W2.T, b2.reshape(_H2, 1),
        Wout[:_H2].T, Wout[_H2:].T, bout.reshape(_NCLS, 1))
    return outT.T


# speedup vs baseline: 15.9048x; 1.0636x over previous
import functools

import jax
import jax.numpy as jnp
from jax import lax
from jax.experimental import pallas as pl
from jax.experimental.pallas import tpu as pltpu
from jax.experimental.pallas import tpu_sc as plsc

_B = 16384
_NF = 26
_NC = 13
_V = 100000
_D = 16
_WD = 128
_H1 = 256
_H2 = 128
_NCLS = 2
_DE = _NF * _D

_NCORES = 2
_NSUB = 16
_NW = _NCORES * _NSUB
_PPW = _DE // _NW
_BCH = 8192
_NBCH = _B // _BCH


_UNR = 8


def _sc_plane_gather_body(t2_hbm, idxT_hbm, out_hbm, plane_v, iv_v, ob_v):
    wid = lax.axis_index("s") * _NCORES + lax.axis_index("c")

    def plane_step(p, prev_f):
        plane = wid * _PPW + p
        f = plane // _D
        d = plane % _D

        @pl.when(f != prev_f)
        def _():
            pltpu.sync_copy(idxT_hbm.at[f, :], iv_v)

        pltpu.sync_copy(t2_hbm.at[f, d, :], plane_v)

        def bchunk(c, carry2):
            def step(i, carry3):
                base = i * (16 * _UNR)
                for j in range(_UNR):
                    o = base + j * 16
                    ivv = iv_v[pl.ds(c * _BCH + o, 16)]
                    ob_v[pl.ds(o, 16)] = plsc.load_gather(plane_v, [ivv])
                return carry3

            lax.fori_loop(0, _BCH // (16 * _UNR), step, 0)
            pltpu.sync_copy(ob_v, out_hbm.at[plane, pl.ds(c * _BCH, _BCH)])
            return carry2

        lax.fori_loop(0, _NBCH, bchunk, 0)
        return f

    lax.fori_loop(0, _PPW, plane_step, -1)


@functools.cache
def _sc_gather():
    return pl.kernel(
        _sc_plane_gather_body,
        out_type=jax.ShapeDtypeStruct((_DE, _B), jnp.float32),
        mesh=plsc.VectorSubcoreMesh(core_axis_name="c", subcore_axis_name="s",
                                    num_cores=_NCORES, num_subcores=_NSUB),
        scratch_types=[
            pltpu.VMEM((_V,), jnp.float32),
            pltpu.VMEM((_B,), jnp.int32),
            pltpu.VMEM((_BCH,), jnp.float32),
        ],
        compiler_params=pltpu.CompilerParams(use_tc_tiling_on_sc=True,
                                             needs_layout_passes=False),
    )


def _mlp_body(xdT_ref, embT_ref, xw_ref, w1aT_ref, w1eT_ref, rT_ref, b1_ref,
              w2T_ref, b2_ref, wdT_ref, wwT_ref, boT_ref, outT_ref):
    xdT = xdT_ref[...]
    contT = xdT[_NF:, :].astype(jnp.float32)
    maskT = (xdT[:_NF, :] == 0).astype(jnp.float32)
    cnegT = -jnp.dot(w1aT_ref[...], rT_ref[...],
                     preferred_element_type=jnp.float32)
    x1 = jnp.dot(w1aT_ref[...], embT_ref[...],
                 preferred_element_type=jnp.float32)
    x1 = x1 + jnp.dot(w1eT_ref[...], contT, preferred_element_type=jnp.float32)
    x1 = x1 + jnp.dot(cnegT, maskT, preferred_element_type=jnp.float32)
    x1 = jnp.maximum(x1 + b1_ref[...], 0.0)
    x2 = jnp.dot(w2T_ref[...], x1, preferred_element_type=jnp.float32)
    x2 = jnp.maximum(x2 + b2_ref[...], 0.0)
    o = jnp.dot(wdT_ref[...], x2, preferred_element_type=jnp.float32)
    o = o + jnp.dot(wwT_ref[...], xw_ref[...].T,
                    preferred_element_type=jnp.float32)
    outT_ref[...] = o + boT_ref[...]


_BM = 2048


def _mlp_call(interpret=False):
    full = lambda a, b: pl.BlockSpec((a, b), lambda i: (0, 0))
    return pl.pallas_call(
        _mlp_body,
        grid=(_B // _BM,),
        in_specs=[
            pl.BlockSpec((_NF + _NC, _BM), lambda i: (0, i)),
            pl.BlockSpec((_DE, _BM), lambda i: (0, i)),
            pl.BlockSpec((_BM, _WD), lambda i: (i, 0)),
            full(_H1, _DE),
            full(_H1, _NC),
            full(_DE, _NF),
            full(_H1, 1),
            full(_H2, _H1),
            full(_H2, 1),
            full(_NCLS, _H2),
            full(_NCLS, _WD),
            full(_NCLS, 1),
        ],
        out_specs=pl.BlockSpec((_NCLS, _BM), lambda i: (0, i)),
        out_shape=jax.ShapeDtypeStruct((_NCLS, _B), jnp.float32),
        interpret=interpret,
    )


def kernel(X_w, X_d, tables, W1, b1, W2, b2, Wout, bout):
    t2 = jnp.transpose(tables, (0, 2, 1))
    xdT = X_d.astype(jnp.int32).T
    embT = _sc_gather()(t2, xdT)

    rows0 = tables[:, 0, :]
    R = (jnp.eye(_NF, dtype=jnp.float32)[:, :, None]
         * rows0[:, None, :]).reshape(_NF, _DE)
    outT = _mlp_call()(
        xdT, embT, X_w.astype(jnp.float32),
        W1[:_DE].T, W1[_DE:].T, R.T,
        b1.reshape(_H1, 1), W2.T, b2.reshape(_H2, 1),
        Wout[:_H2].T, Wout[_H2:].T, bout.reshape(_NCLS, 1))
    return outT.T
